# baseline (speedup 1.0000x reference)
import functools

import jax
import jax.numpy as jnp
from jax import lax
from jax.experimental import pallas as pl
from jax.experimental.pallas import tpu as pltpu
from jax.experimental.pallas import tpu_sc as plsc

VOCAB = 1000000
EMBED = 32
NUM_CLASSES = 2
BATCH = 16384

NC = 2
NS = 16
NW = NC * NS
B_PER_W = BATCH // NW


def _sc_gather(embedding, word_ids):
    mesh = plsc.VectorSubcoreMesh(core_axis_name="c", subcore_axis_name="s")

    @functools.partial(
        pl.kernel,
        mesh=mesh,
        out_type=jax.ShapeDtypeStruct((BATCH, EMBED), jnp.float32),
        compiler_params=pltpu.CompilerParams(use_tc_tiling_on_sc=False),
        scratch_types=[
            pltpu.VMEM((B_PER_W,), jnp.int32),
            pltpu.VMEM((B_PER_W, EMBED), jnp.float32),
            pltpu.SemaphoreType.DMA,
        ],
    )
    def k(table_hbm, idx_hbm, out_hbm, idx_v, rows_v, sem):
        wid = lax.axis_index("s") * NC + lax.axis_index("c")
        base = wid * B_PER_W
        pltpu.sync_copy(idx_hbm.at[pl.ds(base, B_PER_W)], idx_v)
        pltpu.async_copy(table_hbm.at[idx_v], rows_v, sem).wait()
        pltpu.sync_copy(rows_v, out_hbm.at[pl.ds(base, B_PER_W)])

    return k(embedding, word_ids)


def _linear_body(feat_ref, wt_ref, b_ref, out_ref):
    out_ref[...] = (
        jnp.dot(feat_ref[...], wt_ref[...], preferred_element_type=jnp.float32)
        + b_ref[...]
    )


def _tc_linear(features, Wt, b2):
    return pl.pallas_call(
        _linear_body,
        out_shape=jax.ShapeDtypeStruct((BATCH, NUM_CLASSES), jnp.float32),
    )(features, Wt, b2)


def kernel(word_ids, embedding, W, b):
    features = _sc_gather(embedding, word_ids)
    return _tc_linear(features, W.T, b.reshape(1, NUM_CLASSES))


# speedup vs baseline: 2.3060x; 2.3060x over previous
import functools

import jax
import jax.numpy as jnp
from jax import lax
from jax.experimental import pallas as pl
from jax.experimental.pallas import tpu as pltpu
from jax.experimental.pallas import tpu_sc as plsc

VOCAB = 1000000
EMBED = 32
NUM_CLASSES = 2
BATCH = 16384

NC = 2
NS = 16
NW = NC * NS
B_PER_W = BATCH // NW
K = 32
NCH = B_PER_W // K
L = 16


def _sc_fused(table, word_ids, W, b):
    mesh = plsc.VectorSubcoreMesh(core_axis_name="c", subcore_axis_name="s")

    @functools.partial(
        pl.kernel,
        mesh=mesh,
        out_type=jax.ShapeDtypeStruct((BATCH * NUM_CLASSES,), jnp.float32),
        compiler_params=pltpu.CompilerParams(needs_layout_passes=False),
        scratch_types=[
            pltpu.VMEM((B_PER_W,), jnp.int32),
                        pltpu.VMEM((K, 8, EMBED), jnp.float32),
                        pltpu.VMEM((B_PER_W * NUM_CLASSES,), jnp.float32),
            pltpu.VMEM((NUM_CLASSES, EMBED), jnp.float32),
                        pltpu.VMEM((L,), jnp.float32),
            pltpu.VMEM((L,), jnp.float32),
            pltpu.SemaphoreType.DMA,
        ],
    )
    def k(tab_hbm, idx_hbm, w_hbm, b_hbm, out_hbm,
          idx_v, feat_v, out_v, w_v, b_v, b_pat_v, sem):
        wid = lax.axis_index("s") * NC + lax.axis_index("c")
        base = wid * B_PER_W
        pltpu.sync_copy(idx_hbm.at[pl.ds(base, B_PER_W)], idx_v)
        pltpu.sync_copy(w_hbm, w_v)
        pltpu.sync_copy(b_hbm, b_pat_v)

        w0a = w_v[0, pl.ds(0, L)]
        w0b = w_v[0, pl.ds(L, L)]
        w1a = w_v[1, pl.ds(0, L)]
        w1b = w_v[1, pl.ds(L, L)]
        b_pat = b_pat_v[...]
        lane = lax.iota(jnp.int32, L)
        lmask = [lane == i for i in range(L)]

        def chunk_body(ci, _):
            cps = []
            ivs = []
            for u in range(K // L):
                iv = idx_v[pl.ds(ci * K + u * L, L)]
                ivs.append(iv)
                for t in range(L):
                    g = lax.shift_right_logical(iv[t], 3)
                    cps.append(
                        pltpu.async_copy(tab_hbm.at[g], feat_v.at[u * L + t], sem))
            for cp in cps:
                cp.wait()
            for u in range(K // L):
                iv = ivs[u]
                for half in range(2):
                    acc = b_pat
                    for p in range(8):
                        t = half * 8 + p
                        j = u * L + t
                        s = iv[t] & 7
                        ra = feat_v[j, s, pl.ds(0, L)]
                        rb = feat_v[j, s, pl.ds(L, L)]
                        l0 = jnp.sum(ra * w0a + rb * w0b)
                        l1 = jnp.sum(ra * w1a + rb * w1b)
                        acc = jnp.where(lmask[2 * p], l0, acc)
                        acc = jnp.where(lmask[2 * p + 1], l1, acc)
                    o0 = (ci * K + u * L + half * 8) * 2
                    out_v[pl.ds(o0, L)] = acc + b_pat
            return ()

        lax.fori_loop(0, NCH, chunk_body, ())
        pltpu.sync_copy(out_v, out_hbm.at[pl.ds(base * NUM_CLASSES, B_PER_W * NUM_CLASSES)])

    return k(table, word_ids, W, jnp.tile(b, L // NUM_CLASSES))


def kernel(word_ids, embedding, W, b):
    table3 = embedding.reshape(VOCAB // 8, 8, EMBED)
    flat = _sc_fused(table3, word_ids, W, b)
    return flat.reshape(BATCH, NUM_CLASSES)
